# baseline (speedup 1.0000x reference)
import jax
import jax.numpy as jnp
from jax import lax
from jax.experimental import pallas as pl
from jax.experimental.pallas import tpu as pltpu
from jax.experimental.pallas import tpu_sc as plsc

_N = 8388608
_T = _N // 128
_NC = 2
_NS = 16
_NW = _NC * _NS
_TW = _T // _NW
_CT = 128
_NCHUNK = _TW // _CT
_L = 16


def _sc_body(in_hbm, out_hbm,
             in_buf0, in_buf1, out_buf0, out_buf1,
             sem_in0, sem_in1, sem_out0, sem_out1):
    wid = lax.axis_index("s") * _NC + lax.axis_index("c")
    base = wid * _TW
    in_bufs = (in_buf0, in_buf1)
    out_bufs = (out_buf0, out_buf1)
    sem_ins = (sem_in0, sem_in1)
    sem_outs = (sem_out0, sem_out1)

    def src(c):
        return in_hbm.at[pl.ds(base + c * _CT, _CT)]

    def dst(c):
        return out_hbm.at[pl.ds(base + c * _CT, _CT)]

    pltpu.async_copy(src(0), in_buf0, sem_in0)

    def outer(p, carry):
        for b in range(2):
            c = 2 * p + b
            nb = 1 - b
            @pl.when(c + 1 < _NCHUNK)
            def _():
                pltpu.async_copy(src(c + 1), in_bufs[nb], sem_ins[nb])
            pltpu.make_async_copy(src(c), in_bufs[b], sem_ins[b]).wait()
            @pl.when(c >= 2)
            def _():
                pltpu.make_async_copy(out_bufs[b], dst(c - 2), sem_outs[b]).wait()

            in_buf = in_bufs[b]
            out_buf = out_bufs[b]

            def step(t, carry2):
                for j in range(8):
                    xs = in_buf[t, 0, pl.ds(j * _L, _L)]
                    ys = in_buf[t, 1, pl.ds(j * _L, _L)]
                    xi = xs.astype(jnp.int32)
                    yi = ys.astype(jnp.int32)
                    out_buf[t, pl.ds(j * _L, _L)] = (
                        jnp.right_shift(xi, 4)
                        + jnp.left_shift(jnp.right_shift(yi, 4), 5))
                return carry2

            lax.fori_loop(0, _CT, step, 0, unroll=4)
            pltpu.async_copy(out_buf, dst(c), sem_outs[b])
        return carry

    lax.fori_loop(0, _NCHUNK // 2, outer, 0)
    pltpu.make_async_copy(out_buf0, dst(_NCHUNK - 2), sem_out0).wait()
    pltpu.make_async_copy(out_buf1, dst(_NCHUNK - 1), sem_out1).wait()


@jax.jit
def kernel(stroke_coords):
    a3 = stroke_coords.reshape(_T, 128, 2).transpose(0, 2, 1)
    mesh = plsc.VectorSubcoreMesh(core_axis_name="c", subcore_axis_name="s")
    fn = pl.kernel(
        _sc_body,
        out_type=jax.ShapeDtypeStruct((_T, 128), jnp.int32),
        mesh=mesh,
        scratch_types=[
            pltpu.VMEM((_CT, 2, 128), jnp.float32),
            pltpu.VMEM((_CT, 2, 128), jnp.float32),
            pltpu.VMEM((_CT, 128), jnp.int32),
            pltpu.VMEM((_CT, 128), jnp.int32),
            pltpu.SemaphoreType.DMA,
            pltpu.SemaphoreType.DMA,
            pltpu.SemaphoreType.DMA,
            pltpu.SemaphoreType.DMA,
        ],
        compiler_params=pltpu.CompilerParams(
            use_tc_tiling_on_sc=False,
            needs_layout_passes=False),
    )
    return fn(a3).reshape(_N)
